# Initial kernel scaffold; baseline (speedup 1.0000x reference)
import functools

import jax
import jax.numpy as jnp
from jax import lax
from jax.experimental import pallas as pl
from jax.experimental.pallas import tpu as pltpu
from jax.experimental.pallas import tpu_sc as plsc

_CHUNK = 128


@jax.jit
def _gather_rows(weight, idx2d):
    n_rows, _ = idx2d.shape
    B = n_rows * _CHUNK
    V, D = weight.shape

    mesh = plsc.VectorSubcoreMesh(core_axis_name="c", subcore_axis_name="s")
    NC = mesh.num_cores
    NS = mesh.num_subcores
    NW = NC * NS
    assert n_rows % NW == 0
    rows_per_w = n_rows // NW

    @functools.partial(
        pl.kernel,
        out_type=jax.ShapeDtypeStruct((B, D), jnp.float32),
        mesh=mesh,
        scratch_types=[
            pltpu.VMEM((rows_per_w, _CHUNK), jnp.int32),
            pltpu.VMEM((2, _CHUNK, D), jnp.float32),
            pltpu.SemaphoreType.DMA,
            pltpu.SemaphoreType.DMA,
            pltpu.SemaphoreType.DMA,
        ],
    )
    def k(table_hbm, idx_hbm, out_hbm, idx_v, rows_v, isem, gsem, wsem):
        wid = lax.axis_index("s") * NC + lax.axis_index("c")
        row_base = wid * rows_per_w
        out_base = row_base * _CHUNK

        pltpu.async_copy(
            idx_hbm.at[pl.ds(row_base, rows_per_w)], idx_v, isem
        ).wait()

        def body(j, _):
            slot = lax.rem(j, 2)
            pltpu.async_copy(
                table_hbm.at[idx_v.at[j]], rows_v.at[slot], gsem
            ).wait()
            pltpu.async_copy(
                rows_v.at[slot],
                out_hbm.at[pl.ds(out_base + j * _CHUNK, _CHUNK)],
                wsem,
            ).wait()
            return 0

        lax.fori_loop(0, rows_per_w, body, 0)

    return k(weight, idx2d)


def kernel(token_ids, weight):
    Bt, S = token_ids.shape
    V, D = weight.shape
    idx = token_ids.reshape(-1).astype(jnp.int32)
    idx2d = idx.reshape(-1, _CHUNK)
    out = _gather_rows(weight, idx2d)
    return out.reshape(Bt, S, D)


# speedup vs baseline: 2.9637x; 2.9637x over previous
import functools

import jax
import jax.numpy as jnp
from jax import lax
from jax.experimental import pallas as pl
from jax.experimental.pallas import tpu as pltpu
from jax.experimental.pallas import tpu_sc as plsc

_CHUNK = 128


@jax.jit
def _gather_rows(weight, idx3d):
    NW, rows_per_w, _ = idx3d.shape
    B = NW * rows_per_w * _CHUNK
    V, D = weight.shape

    mesh = plsc.VectorSubcoreMesh(core_axis_name="c", subcore_axis_name="s")
    NC = mesh.num_cores
    NS = mesh.num_subcores
    assert NW == NC * NS

    @functools.partial(
        pl.kernel,
        out_type=jax.ShapeDtypeStruct((B, D), jnp.float32),
        mesh=mesh,
        scratch_types=[
            pltpu.VMEM((rows_per_w, _CHUNK), jnp.int32),
            pltpu.VMEM((2, _CHUNK, D), jnp.float32),
            pltpu.SemaphoreType.DMA,
            pltpu.SemaphoreType.DMA,
            pltpu.SemaphoreType.DMA,
        ],
    )
    def k(table_hbm, idx_hbm, out_hbm, idx_v, rows_v, isem, gsem, wsem):
        wid = lax.axis_index("s") * NC + lax.axis_index("c")
        out_base = wid * rows_per_w * _CHUNK

        pltpu.async_copy(idx_hbm.at[wid], idx_v, isem).wait()

        def body(j, _):
            slot = lax.rem(j, 2)
            pltpu.async_copy(
                table_hbm.at[idx_v.at[j]], rows_v.at[slot], gsem
            ).wait()
            pltpu.async_copy(
                rows_v.at[slot],
                out_hbm.at[pl.ds(out_base + j * _CHUNK, _CHUNK)],
                wsem,
            ).wait()
            return 0

        lax.fori_loop(0, rows_per_w, body, 0)

    return k(weight, idx3d)


def kernel(token_ids, weight):
    Bt, S = token_ids.shape
    V, D = weight.shape
    idx = token_ids.reshape(-1).astype(jnp.int32)
    idx3d = idx.reshape(32, -1, _CHUNK)
    out = _gather_rows(weight, idx3d)
    return out.reshape(Bt, S, D)


# speedup vs baseline: 3.3278x; 1.1229x over previous
import functools

import jax
import jax.numpy as jnp
from jax import lax
from jax.experimental import pallas as pl
from jax.experimental.pallas import tpu as pltpu
from jax.experimental.pallas import tpu_sc as plsc

_CHUNK = 128


@jax.jit
def _gather_rows(weight, idx3d):
    NW, rows_per_w, _ = idx3d.shape
    B = NW * rows_per_w * _CHUNK
    V, D = weight.shape

    mesh = plsc.VectorSubcoreMesh(core_axis_name="c", subcore_axis_name="s")
    NC = mesh.num_cores
    NS = mesh.num_subcores
    assert NW == NC * NS

    @functools.partial(
        pl.kernel,
        out_type=jax.ShapeDtypeStruct((B, D), jnp.float32),
        mesh=mesh,
        scratch_types=[
            pltpu.VMEM((rows_per_w, _CHUNK), jnp.int32),
            pltpu.VMEM((2, _CHUNK, D), jnp.float32),
            pltpu.SemaphoreType.DMA,
            pltpu.SemaphoreType.DMA,
            pltpu.SemaphoreType.DMA,
        ],
    )
    def k(table_hbm, idx_hbm, out_hbm, idx_v, rows_v, isem, gsem, wsem):
        wid = lax.axis_index("s") * NC + lax.axis_index("c")
        out_base = wid * rows_per_w * _CHUNK

        pltpu.async_copy(idx_hbm.at[wid], idx_v, isem).wait()

        def fire_gather(j):
            pltpu.async_copy(
                table_hbm.at[idx_v.at[j]], rows_v.at[lax.rem(j, 2)], gsem
            )

        def out_slice(j):
            return out_hbm.at[pl.ds(out_base + j * _CHUNK, _CHUNK)]

        fire_gather(0)

        def body(j, _):
            slot = lax.rem(j, 2)

            @pl.when(j >= 1)
            def _():
                pltpu.make_async_copy(
                    rows_v.at[1 - slot], out_slice(j), wsem
                ).wait()

            @pl.when(j + 1 < rows_per_w)
            def _():
                fire_gather(j + 1)

            pltpu.make_async_copy(
                table_hbm.at[idx_v.at[j]], rows_v.at[slot], gsem
            ).wait()
            pltpu.async_copy(rows_v.at[slot], out_slice(j), wsem)
            return 0

        lax.fori_loop(0, rows_per_w, body, 0)
        pltpu.make_async_copy(
            rows_v.at[lax.rem(rows_per_w - 1, 2)],
            out_slice(rows_per_w - 1),
            wsem,
        ).wait()

    return k(weight, idx3d)


def kernel(token_ids, weight):
    Bt, S = token_ids.shape
    V, D = weight.shape
    idx = token_ids.reshape(-1).astype(jnp.int32)
    idx3d = idx.reshape(32, -1, _CHUNK)
    out = _gather_rows(weight, idx3d)
    return out.reshape(Bt, S, D)


# speedup vs baseline: 3.3552x; 1.0082x over previous
import functools

import jax
import jax.numpy as jnp
from jax import lax
from jax.experimental import pallas as pl
from jax.experimental.pallas import tpu as pltpu
from jax.experimental.pallas import tpu_sc as plsc

_CHUNK = 128
_NBUF = 4


@jax.jit
def _gather_rows(weight, idx3d):
    NW, rows_per_w, _ = idx3d.shape
    B = NW * rows_per_w * _CHUNK
    V, D = weight.shape

    mesh = plsc.VectorSubcoreMesh(core_axis_name="c", subcore_axis_name="s")
    NC = mesh.num_cores
    NS = mesh.num_subcores
    assert NW == NC * NS

    @functools.partial(
        pl.kernel,
        out_type=jax.ShapeDtypeStruct((B, D), jnp.float32),
        mesh=mesh,
        scratch_types=[
            pltpu.VMEM((rows_per_w, _CHUNK), jnp.int32),
            pltpu.VMEM((_NBUF, _CHUNK, D), jnp.float32),
            pltpu.SemaphoreType.DMA,
            pltpu.SemaphoreType.DMA,
            pltpu.SemaphoreType.DMA,
        ],
    )
    def k(table_hbm, idx_hbm, out_hbm, idx_v, rows_v, isem, gsem, wsem):
        wid = lax.axis_index("s") * NC + lax.axis_index("c")
        out_base = wid * rows_per_w * _CHUNK

        pltpu.async_copy(idx_hbm.at[wid], idx_v, isem).wait()

        def fire_gather(j):
            pltpu.async_copy(
                table_hbm.at[idx_v.at[j]], rows_v.at[lax.rem(j, _NBUF)], gsem
            )

        def out_slice(j):
            return out_hbm.at[pl.ds(out_base + j * _CHUNK, _CHUNK)]

        for p in range(_NBUF - 1):
            fire_gather(p)

        def body(j, _):
            slot = lax.rem(j, _NBUF)

            @pl.when(j >= 1)
            def _():
                pltpu.make_async_copy(
                    rows_v.at[slot], out_slice(j), wsem
                ).wait()

            @pl.when(j + _NBUF - 1 < rows_per_w)
            def _():
                fire_gather(j + _NBUF - 1)

            pltpu.make_async_copy(
                table_hbm.at[idx_v.at[j]], rows_v.at[slot], gsem
            ).wait()
            pltpu.async_copy(rows_v.at[slot], out_slice(j), wsem)
            return 0

        lax.fori_loop(0, rows_per_w, body, 0)
        pltpu.make_async_copy(
            rows_v.at[lax.rem(rows_per_w - 1, _NBUF)],
            out_slice(rows_per_w - 1),
            wsem,
        ).wait()

    return k(weight, idx3d)


def kernel(token_ids, weight):
    Bt, S = token_ids.shape
    V, D = weight.shape
    idx = token_ids.reshape(-1).astype(jnp.int32)
    idx3d = idx.reshape(32, -1, _CHUNK)
    out = _gather_rows(weight, idx3d)
    return out.reshape(Bt, S, D)
